# baseline (speedup 1.0000x reference)
import functools

import jax
import jax.numpy as jnp
from jax import lax
from jax.experimental import pallas as pl
from jax.experimental.pallas import tpu as pltpu
from jax.experimental.pallas import tpu_sc as plsc

N = 10000
D_IN = 128
D_HID = 128
D_OUT = 64
E = 320000

NC = 2
NS = 16
L = 16

N_PAD = 10240
DUMMY = N
E_CH = 640
T_EDG = 16
E_PAD = NC * NS * T_EDG * E_CH

_MESH = dict(core_axis_name="c", subcore_axis_name="s", num_cores=NC,
             num_subcores=NS)


def _gat_body(y_hbm, src_hbm, out_hbm, idx_v, rows_v, sem):
    c = lax.axis_index("c")
    s = lax.axis_index("s")
    w = s * NC + c

    def chunk(t, carry):
        pltpu.sync_copy(src_hbm.at[w, t], idx_v)
        pltpu.async_copy(y_hbm.at[idx_v], rows_v, sem).wait()
        pltpu.sync_copy(rows_v, out_hbm.at[w, t])
        return carry
    lax.fori_loop(0, T_EDG, chunk, 0)


_gat = functools.partial(
    pl.kernel,
    out_type=jax.ShapeDtypeStruct((NC * NS, T_EDG, E_CH, D_HID),
                                  jnp.float32),
    mesh=plsc.VectorSubcoreMesh(**_MESH),
    scratch_types=[
        pltpu.VMEM((E_CH,), jnp.int32),
        pltpu.VMEM((E_CH, D_HID), jnp.float32),
        pltpu.SemaphoreType.DMA,
    ],
)(_gat_body)


_MB = 2000


def _m1s_body(x_ref, w_ref, h_ref, y_ref, d_ref):
    deg = jnp.sum(h_ref[...], axis=1, keepdims=True) + 1.0
    dinv = lax.rsqrt(deg)
    y_ref[...] = jnp.dot(x_ref[...], w_ref[...],
                         preferred_element_type=jnp.float32) * dinv
    d_ref[...] = dinv


_m1s = pl.pallas_call(
    _m1s_body,
    grid=(N // _MB,),
    in_specs=[
        pl.BlockSpec((_MB, D_IN), lambda i: (i, 0)),
        pl.BlockSpec((D_IN, D_HID), lambda i: (0, 0)),
        pl.BlockSpec((_MB, NC), lambda i: (i, 0)),
    ],
    out_specs=[
        pl.BlockSpec((_MB, D_HID), lambda i: (i, 0)),
        pl.BlockSpec((_MB, 1), lambda i: (i, 0)),
    ],
    out_shape=[
        jax.ShapeDtypeStruct((N, D_HID), jnp.float32),
        jax.ShapeDtypeStruct((N, 1), jnp.float32),
    ],
)


def _m2s_body(a_ref, d_ref, w_ref, y_ref):
    dinv = d_ref[...]
    h = jnp.maximum(a_ref[...] * dinv, 0.0)
    y_ref[...] = jnp.dot(h, w_ref[...],
                         preferred_element_type=jnp.float32) * dinv


_m2s = pl.pallas_call(
    _m2s_body,
    grid=(N // _MB,),
    in_specs=[
        pl.BlockSpec((_MB, D_HID), lambda i: (i, 0)),
        pl.BlockSpec((_MB, 1), lambda i: (i, 0)),
        pl.BlockSpec((D_HID, D_OUT), lambda i: (0, 0)),
    ],
    out_specs=pl.BlockSpec((_MB, D_OUT), lambda i: (i, 0)),
    out_shape=jax.ShapeDtypeStruct((N, D_OUT), jnp.float32),
)


def _m3s_body(a_ref, d_ref, o_ref):
    o = a_ref[...] * d_ref[...]
    m = jnp.max(o, axis=1, keepdims=True)
    o_ref[...] = o - m - jnp.log(jnp.sum(jnp.exp(o - m), axis=1,
                                         keepdims=True))


_m3s = pl.pallas_call(
    _m3s_body,
    grid=(N // _MB,),
    in_specs=[
        pl.BlockSpec((_MB, D_OUT), lambda i: (i, 0)),
        pl.BlockSpec((_MB, 1), lambda i: (i, 0)),
    ],
    out_specs=pl.BlockSpec((_MB, D_OUT), lambda i: (i, 0)),
    out_shape=jax.ShapeDtypeStruct((N, D_OUT), jnp.float32),
)


def kernel(x, edge_index, W1, W2):
    src = edge_index[0].astype(jnp.int32)
    dst = edge_index[1].astype(jnp.int32)
    pad = jnp.full((E_PAD - E,), DUMMY, jnp.int32)
    src_g = jnp.concatenate([src, pad]).reshape(NC * NS, T_EDG, E_CH)
    dst_flat = jnp.concatenate([dst, pad])

    degj = jnp.zeros((N,), jnp.float32).at[dst].add(1.0)
    hist2 = jnp.stack([degj, jnp.zeros_like(degj)], axis=1)

    def _scatter(y, msgs):
        return y + jnp.zeros_like(y).at[dst_flat].add(
            msgs.reshape(E_PAD, -1)[:, :y.shape[1]], mode="drop")

    zpad = jnp.zeros((N_PAD - N, D_HID), jnp.float32)

    y1, dinv = _m1s(x, W1, hist2)
    msgs1 = _gat(jnp.concatenate([y1, zpad]), src_g)
    acc1 = _scatter(y1, msgs1)

    y2 = _m2s(acc1, dinv, W2)
    y2w = jnp.concatenate(
        [y2, jnp.zeros((N, D_HID - D_OUT), jnp.float32)], axis=1)
    msgs2 = _gat(jnp.concatenate([y2w, zpad]), src_g)
    acc2 = _scatter(y2, msgs2)

    return _m3s(acc2, dinv)
